# Initial kernel scaffold; baseline (speedup 1.0000x reference)
import functools

import jax
import jax.numpy as jnp
from jax import lax
from jax.experimental import pallas as pl
from jax.experimental.pallas import tpu as pltpu
from jax.experimental.pallas import tpu_sc as plsc

D_MODEL = 32
NC, NS = 2, 16
NW = NC * NS
SUB = 128
NSUB = 10
CHUNK = SUB * NSUB


def _embed_body(idx_hbm, tab_hbm, out_hbm, idx_v, rows_v, gsem, *,
                rows_per_w, nchunk):
    wid = lax.axis_index("s") * NC + lax.axis_index("c")
    r0 = wid * rows_per_w

    @pl.loop(0, nchunk)
    def chunk_body(g):
        rbase = r0 + g * NSUB
        pltpu.sync_copy(idx_hbm.at[pl.ds(rbase, NSUB)], idx_v)
        for j in range(NSUB):
            pltpu.async_copy(tab_hbm.at[idx_v.at[j]], rows_v.at[j], gsem)
        for j in range(NSUB):
            pltpu.make_async_copy(tab_hbm.at[idx_v.at[j]], rows_v.at[j],
                                  gsem).wait()
        pltpu.sync_copy(rows_v, out_hbm.at[pl.ds(rbase, NSUB)])


def kernel(tokens, weights):
    orig_shape = tokens.shape
    b = tokens.size
    assert b % (NW * SUB) == 0
    rows_per_w = b // (NW * SUB)
    assert rows_per_w % NSUB == 0
    nchunk = rows_per_w // NSUB

    idx2d = tokens.reshape(b // SUB, SUB).astype(jnp.int32)

    mesh = plsc.VectorSubcoreMesh(core_axis_name="c", subcore_axis_name="s")
    grid_fn = pl.kernel(
        functools.partial(_embed_body, rows_per_w=rows_per_w, nchunk=nchunk),
        out_type=jax.ShapeDtypeStruct((b // SUB, SUB, D_MODEL), jnp.float32),
        mesh=mesh,
        scratch_types=[
            pltpu.VMEM((NSUB, SUB), jnp.int32),
            pltpu.VMEM((NSUB, SUB, D_MODEL), jnp.float32),
            pltpu.SemaphoreType.DMA,
        ],
    )
    out = grid_fn(idx2d, weights)
    return out.reshape(*orig_shape, D_MODEL)


# speedup vs baseline: 1.4579x; 1.4579x over previous
import functools

import jax
import jax.numpy as jnp
from jax import lax
from jax.experimental import pallas as pl
from jax.experimental.pallas import tpu as pltpu
from jax.experimental.pallas import tpu_sc as plsc

D_MODEL = 32
NC, NS = 2, 16
NW = NC * NS
SUB = 128
NSUB = 8
CHUNK = SUB * NSUB


def _embed_body(idx_hbm, tab_hbm, out_hbm, idx_v, rows_v, gsem, *,
                rows_per_w, nchunk):
    wid = lax.axis_index("s") * NC + lax.axis_index("c")
    r0 = wid * rows_per_w

    @pl.loop(0, nchunk)
    def chunk_body(g):
        rbase = r0 + g * NSUB
        pltpu.sync_copy(idx_hbm.at[pl.ds(rbase, NSUB)], idx_v)
        for j in range(NSUB):
            pltpu.async_copy(tab_hbm.at[idx_v.at[j]], rows_v.at[j], gsem)
        for j in range(NSUB):
            pltpu.make_async_copy(tab_hbm.at[idx_v.at[j]], rows_v.at[j],
                                  gsem).wait()
        pltpu.sync_copy(rows_v, out_hbm.at[pl.ds(rbase, NSUB)])


def kernel(tokens, weights):
    orig_shape = tokens.shape
    b = tokens.size
    assert b % (NW * SUB) == 0
    rows_per_w = b // (NW * SUB)
    assert rows_per_w % NSUB == 0
    nchunk = rows_per_w // NSUB

    idx2d = tokens.reshape(b // SUB, SUB).astype(jnp.int32)

    mesh = plsc.VectorSubcoreMesh(core_axis_name="c", subcore_axis_name="s")
    grid_fn = pl.kernel(
        functools.partial(_embed_body, rows_per_w=rows_per_w, nchunk=nchunk),
        out_type=jax.ShapeDtypeStruct((b // SUB, SUB, D_MODEL), jnp.float32),
        mesh=mesh,
        scratch_types=[
            pltpu.VMEM((NSUB, SUB), jnp.int32),
            pltpu.VMEM((NSUB, SUB, D_MODEL), jnp.float32),
            pltpu.SemaphoreType.DMA,
        ],
        compiler_params=pltpu.CompilerParams(use_tc_tiling_on_sc=False),
    )
    out = grid_fn(idx2d, weights)
    return out.reshape(*orig_shape, D_MODEL)


# speedup vs baseline: 1.4849x; 1.0185x over previous
import functools

import jax
import jax.numpy as jnp
from jax import lax
from jax.experimental import pallas as pl
from jax.experimental.pallas import tpu as pltpu
from jax.experimental.pallas import tpu_sc as plsc

D_MODEL = 32
NC, NS = 2, 16
NW = NC * NS
SUB = 128
NSUB = 8
CHUNK = SUB * NSUB


def _embed_body(idx_hbm, tab_hbm, out_hbm, idx_v, rows_v, gsem0, gsem1, osem,
                *, rows_per_w, nchunk):
    wid = lax.axis_index("s") * NC + lax.axis_index("c")
    r0 = wid * rows_per_w
    gsems = (gsem0, gsem1)

    def load_and_fire(g, slot):
        pltpu.sync_copy(idx_hbm.at[pl.ds(r0 + g * NSUB, NSUB)], idx_v.at[slot])
        for j in range(NSUB):
            pltpu.async_copy(tab_hbm.at[idx_v.at[slot, j]],
                             rows_v.at[slot, j], gsems[slot])

    def drain_gathers(slot):
        for j in range(NSUB):
            pltpu.make_async_copy(tab_hbm.at[idx_v.at[slot, j]],
                                  rows_v.at[slot, j], gsems[slot]).wait()

    def write_out(g, slot):
        pltpu.async_copy(rows_v.at[slot], out_hbm.at[pl.ds(r0 + g * NSUB,
                                                           NSUB)], osem)
        pltpu.make_async_copy(rows_v.at[slot],
                              out_hbm.at[pl.ds(r0 + g * NSUB, NSUB)],
                              osem).wait()

    load_and_fire(0, 0)

    npairs = (nchunk - 1) // 2

    @pl.loop(0, npairs)
    def pair_body(i):
        for sub in (0, 1):
            g = 2 * i + sub
            s = sub
            load_and_fire(g + 1, 1 - s)
            drain_gathers(s)
            write_out(g, s)

    g = nchunk - 1
    s = g % 2
    drain_gathers(s)
    write_out(g, s)


def kernel(tokens, weights):
    orig_shape = tokens.shape
    b = tokens.size
    assert b % (NW * SUB) == 0
    rows_per_w = b // (NW * SUB)
    assert rows_per_w % NSUB == 0
    nchunk = rows_per_w // NSUB
    assert nchunk >= 3 and nchunk % 2 == 1

    idx2d = tokens.reshape(b // SUB, SUB).astype(jnp.int32)

    mesh = plsc.VectorSubcoreMesh(core_axis_name="c", subcore_axis_name="s")
    grid_fn = pl.kernel(
        functools.partial(_embed_body, rows_per_w=rows_per_w, nchunk=nchunk),
        out_type=jax.ShapeDtypeStruct((b // SUB, SUB, D_MODEL), jnp.float32),
        mesh=mesh,
        scratch_types=[
            pltpu.VMEM((2, NSUB, SUB), jnp.int32),
            pltpu.VMEM((2, NSUB, SUB, D_MODEL), jnp.float32),
            pltpu.SemaphoreType.DMA,
            pltpu.SemaphoreType.DMA,
            pltpu.SemaphoreType.DMA,
        ],
        compiler_params=pltpu.CompilerParams(use_tc_tiling_on_sc=False),
    )
    out = grid_fn(idx2d, weights)
    return out.reshape(*orig_shape, D_MODEL)
